# baseline (speedup 1.0000x reference)
import functools

import jax
import jax.numpy as jnp
from jax import lax
from jax.experimental import pallas as pl
from jax.experimental.pallas import tpu as pltpu
from jax.experimental.pallas import tpu_sc as plsc

_B, _D, _H, _K = 4096, 512, 512, 16
_LANES = 16


def _build_mt_sparsecore(idx2, w12):
    info = plsc.get_sparse_core_info()
    nw = info.num_cores * info.num_subcores
    th = _H // nw

    mesh = plsc.VectorSubcoreMesh(core_axis_name="c", subcore_axis_name="s")

    @functools.partial(
        pl.kernel,
        mesh=mesh,
        compiler_params=pltpu.CompilerParams(needs_layout_passes=False),
        out_type=jax.ShapeDtypeStruct((_H, _D), jnp.float32),
        scratch_types=[
            pltpu.VMEM((_K, 128), jnp.int32),
            pltpu.VMEM((_K, 128), jnp.float32),
            pltpu.VMEM((th, _D), jnp.float32),
        ],
    )
    def build(idx_hbm, w_hbm, m_hbm, idx_v, w_v, m_v):
        wid = lax.axis_index("s") * info.num_cores + lax.axis_index("c")
        base = wid * th
        colblk = pl.multiple_of((base // 128) * 128, 128)
        pltpu.sync_copy(idx_hbm.at[:, pl.ds(colblk, 128)], idx_v)
        pltpu.sync_copy(w_hbm.at[:, pl.ds(colblk, 128)], w_v)

        zero = jnp.zeros((_LANES,), jnp.float32)

        def zero_chunk(i, c):
            for j in range(th):
                m_v[j, pl.ds(i * _LANES, _LANES)] = zero
            return c

        lax.fori_loop(0, _D // _LANES, zero_chunk, 0)

        lane = lax.broadcasted_iota(jnp.int32, (_LANES,), 0)
        cols = lane + (base - colblk)

        def scatter_k(k, c):
            kvec = lane * 0 + k
            taps = plsc.load_gather(idx_v, [kvec, cols])
            wk = plsc.load_gather(w_v, [kvec, cols])
            plsc.addupdate_scatter(m_v, [lane, taps], wk)
            return c

        lax.fori_loop(0, _K, scatter_k, 0)

        pltpu.sync_copy(m_v, m_hbm.at[pl.ds(base, th), :])

    return build(idx2, w12)


def _forward_body(x_ref, mt_ref, b1_ref, w2_ref, b2_ref, ha_ref, out_ref):
    dims = (((1,), (1,)), ((), ()))
    xf = x_ref[...]
    mf = mt_ref[...]
    xh = xf.astype(jnp.bfloat16)
    mh = mf.astype(jnp.bfloat16)
    xl = (xf - xh.astype(jnp.float32)).astype(jnp.bfloat16)
    ml = (mf - mh.astype(jnp.float32)).astype(jnp.bfloat16)
    h = lax.dot_general(
        xh, mh, dimension_numbers=dims, preferred_element_type=jnp.float32
    )
    h += lax.dot_general(
        xh, ml, dimension_numbers=dims, preferred_element_type=jnp.float32
    )
    h += lax.dot_general(
        xl, mh, dimension_numbers=dims, preferred_element_type=jnp.float32
    )
    ha = jnp.tanh(h + b1_ref[...])
    ha_ref[...] = ha
    o = jnp.sum(ha * w2_ref[...], axis=1, keepdims=True)
    out_ref[...] = jnp.tanh(o + b2_ref[...]).T


def _forward_tensorcore(x, mt, b1, w2, b2):
    bb = 2048
    grid = (_B // bb,)
    ha, out = pl.pallas_call(
        _forward_body,
        grid=grid,
        compiler_params=pltpu.CompilerParams(
            dimension_semantics=("parallel",)
        ),
        in_specs=[
            pl.BlockSpec((bb, _D), lambda i: (i, 0)),
            pl.BlockSpec((_H, _D), lambda i: (0, 0)),
            pl.BlockSpec((1, _H), lambda i: (0, 0)),
            pl.BlockSpec((1, _H), lambda i: (0, 0)),
            pl.BlockSpec((1, 1), lambda i: (0, 0)),
        ],
        out_specs=[
            pl.BlockSpec((bb, _H), lambda i: (i, 0)),
            pl.BlockSpec((1, bb), lambda i: (0, i)),
        ],
        out_shape=[
            jax.ShapeDtypeStruct((_B, _H), jnp.float32),
            jax.ShapeDtypeStruct((1, _B), jnp.float32),
        ],
    )(x, mt, b1.reshape(1, _H), w2.reshape(1, _H), b2.reshape(1, 1))
    return ha, out.reshape(_B)


def kernel(x, indices, W1, b1, W2, b2):
    mt = _build_mt_sparsecore(indices.T, W1.T)
    return _forward_tensorcore(x, mt, b1, W2, b2)


# speedup vs baseline: 1.0214x; 1.0214x over previous
import functools

import jax
import jax.numpy as jnp
from jax import lax
from jax.experimental import pallas as pl
from jax.experimental.pallas import tpu as pltpu
from jax.experimental.pallas import tpu_sc as plsc

_B, _D, _H, _K = 4096, 512, 512, 16
_LANES = 16


def _build_mt_sparsecore(idx2, w12):
    info = plsc.get_sparse_core_info()
    nw = info.num_cores * info.num_subcores
    th = _H // nw

    mesh = plsc.VectorSubcoreMesh(core_axis_name="c", subcore_axis_name="s")

    @functools.partial(
        pl.kernel,
        mesh=mesh,
        compiler_params=pltpu.CompilerParams(needs_layout_passes=False),
        out_type=jax.ShapeDtypeStruct((_H, _D), jnp.float32),
        scratch_types=[
            pltpu.VMEM((_K, 128), jnp.int32),
            pltpu.VMEM((_K, 128), jnp.float32),
            pltpu.VMEM((th, _D), jnp.float32),
        ],
    )
    def build(idx_hbm, w_hbm, m_hbm, idx_v, w_v, m_v):
        wid = lax.axis_index("s") * info.num_cores + lax.axis_index("c")
        base = wid * th
        colblk = pl.multiple_of((base // 128) * 128, 128)
        pltpu.sync_copy(idx_hbm.at[:, pl.ds(colblk, 128)], idx_v)
        pltpu.sync_copy(w_hbm.at[:, pl.ds(colblk, 128)], w_v)

        zero = jnp.zeros((_LANES,), jnp.float32)

        def zero_chunk(i, c):
            for j in range(th):
                m_v[j, pl.ds(i * _LANES, _LANES)] = zero
            return c

        lax.fori_loop(0, _D // _LANES, zero_chunk, 0)

        lane = lax.broadcasted_iota(jnp.int32, (_LANES,), 0)
        cols = lane + (base - colblk)

        def scatter_k(k, c):
            kvec = lane * 0 + k
            taps = plsc.load_gather(idx_v, [kvec, cols])
            wk = plsc.load_gather(w_v, [kvec, cols])
            plsc.addupdate_scatter(m_v, [lane, taps], wk)
            return c

        lax.fori_loop(0, _K, scatter_k, 0)

        pltpu.sync_copy(m_v, m_hbm.at[pl.ds(base, th), :])

    return build(idx2, w12)


def _forward_body(x_ref, mt_ref, b1_ref, w2_ref, b2_ref, ha_ref, out_ref):
    dims = (((1,), (1,)), ((), ()))
    xf = x_ref[...]
    mf = mt_ref[...]
    xh = xf.astype(jnp.bfloat16)
    mh = mf.astype(jnp.bfloat16)
    xl = (xf - xh.astype(jnp.float32)).astype(jnp.bfloat16)
    ml = (mf - mh.astype(jnp.float32)).astype(jnp.bfloat16)
    h = lax.dot_general(
        xh, mh, dimension_numbers=dims, preferred_element_type=jnp.float32
    )
    h += lax.dot_general(
        xh, ml, dimension_numbers=dims, preferred_element_type=jnp.float32
    )
    h += lax.dot_general(
        xl, mh, dimension_numbers=dims, preferred_element_type=jnp.float32
    )
    ha = jnp.tanh(h + b1_ref[...])
    ha_ref[...] = ha
    o = jnp.sum(ha * w2_ref[...], axis=1, keepdims=True)
    out_ref[...] = jnp.tanh(o + b2_ref[...]).T


def _forward_tensorcore(x, mt, b1, w2, b2):
    bb = 1024
    grid = (_B // bb,)
    ha, out = pl.pallas_call(
        _forward_body,
        grid=grid,
        compiler_params=pltpu.CompilerParams(
            dimension_semantics=("parallel",)
        ),
        in_specs=[
            pl.BlockSpec((bb, _D), lambda i: (i, 0)),
            pl.BlockSpec((_H, _D), lambda i: (0, 0)),
            pl.BlockSpec((1, _H), lambda i: (0, 0)),
            pl.BlockSpec((1, _H), lambda i: (0, 0)),
            pl.BlockSpec((1, 1), lambda i: (0, 0)),
        ],
        out_specs=[
            pl.BlockSpec((bb, _H), lambda i: (i, 0)),
            pl.BlockSpec((1, bb), lambda i: (0, i)),
        ],
        out_shape=[
            jax.ShapeDtypeStruct((_B, _H), jnp.float32),
            jax.ShapeDtypeStruct((1, _B), jnp.float32),
        ],
    )(x, mt, b1.reshape(1, _H), w2.reshape(1, _H), b2.reshape(1, 1))
    return ha, out.reshape(_B)


def kernel(x, indices, W1, b1, W2, b2):
    mt = _build_mt_sparsecore(indices.T, W1.T)
    return _forward_tensorcore(x, mt, b1, W2, b2)
